# baseline (speedup 1.0000x reference)
import math

import jax
import jax.numpy as jnp
from jax import lax
from jax.experimental import pallas as pl
from jax.experimental.pallas import tpu as pltpu

_C10 = (((1,), (0,)), ((), ()))
_CLAST = (((1,), (1,)), ((), ()))
_CC00 = (((0,), (0,)), ((), ()))
_C01 = (((0,), (1,)), ((), ()))


def _layernorm_f32(h, g, b, eps=1e-5):
    mu = jnp.mean(h, axis=-1, keepdims=True)
    m2 = jnp.mean(h * h, axis=-1, keepdims=True)
    k = lax.rsqrt(m2 - mu * mu + eps)
    kg = k * g
    return h * kg + (b - mu * kg)


def _make_body(n_layers, n_heads, head_size, seq_len, vocab_size, emb_size, bpp):
    L, nh, hs = n_layers, n_heads, head_size
    T, V, E = seq_len, vocab_size, emb_size
    M = bpp * T
    bf = jnp.bfloat16
    f32 = jnp.float32

    def body(ids_ref, te_ref, pe_ref, wqkv_ref, wproj_ref, g1_ref, b1_ref,
             wf1_ref, bf1_ref, wf2_ref, bf2_ref, g2_ref, b2_ref,
             wv_ref, bv_ref, o_ref, x_scr):
        l = pl.program_id(1)

        @pl.when(l == 0)
        def _embed():
            ids_rep = pltpu.repeat(ids_ref[...], V // 128, axis=1)
            col = lax.broadcasted_iota(jnp.int32, (M, V), 1)
            onehot = jnp.where(ids_rep == col, 1.0, 0.0).astype(bf)
            tok = lax.dot_general(onehot, te_ref[...], _C10,
                                  preferred_element_type=f32)
            x_scr[...] = tok + pltpu.repeat(pe_ref[...], bpp, axis=0)

        x32 = x_scr[...]

        ri = lax.broadcasted_iota(jnp.int32, (T, T), 0)
        ci = lax.broadcasted_iota(jnp.int32, (T, T), 1)
        causal_t = ri <= ci

        xb = x32.astype(bf)
        qkv = lax.dot_general(xb, wqkv_ref[...], _CLAST,
                              preferred_element_type=f32)
        qkvb = qkv.astype(bf)
        projs = []
        for b in range(bpp):
            r0 = b * T
            houts = []
            for h in range(nh):
                q = qkvb[r0:r0 + T, h * hs:(h + 1) * hs]
                k = qkvb[r0:r0 + T, (nh + h) * hs:(nh + h + 1) * hs]
                v = qkvb[r0:r0 + T, (2 * nh + h) * hs:(2 * nh + h + 1) * hs]
                st = lax.dot_general(k, q, _CLAST,
                                     preferred_element_type=f32)
                et = jnp.exp(jnp.where(causal_t, st, -1e30))
                dent = jnp.sum(et, axis=0, keepdims=True)
                pvt = lax.dot_general(v, et.astype(bf), _CC00,
                                      preferred_element_type=f32)
                houts.append(pvt * pl.reciprocal(dent, approx=True))
            projs.append(lax.dot_general(
                jnp.concatenate(houts, axis=0).astype(bf),
                wproj_ref[...], _C01, preferred_element_type=f32))
        proj = jnp.concatenate(projs, axis=0)
        h1 = _layernorm_f32(x32 + proj, g1_ref[...], b1_ref[...])
        a = lax.dot_general(h1.astype(bf), wf1_ref[...], _CLAST,
                            preferred_element_type=f32)
        ab = jnp.maximum((a + bf1_ref[...]).astype(bf), 0)
        f = lax.dot_general(ab, wf2_ref[...], _CLAST,
                            preferred_element_type=f32)
        f = f + bf2_ref[...]
        x_new = _layernorm_f32(f + x32, g2_ref[...], b2_ref[...])
        x_scr[...] = x_new

        @pl.when(l == L - 1)
        def _logits():
            y = lax.dot_general(x_new.astype(bf), wv_ref[...], _CLAST,
                                preferred_element_type=f32)
            y = (y + bv_ref[...]).astype(o_ref.dtype)
            for b in range(bpp):
                o_ref[b] = y[b * T:(b + 1) * T, :]

    return body


def kernel(ids, tok_emb, pos_emb, w_qkv, w_proj, ln1_g, ln1_b,
           w_ff1, b_ff1, w_ff2, b_ff2, ln2_g, ln2_b, w_vocab, b_vocab):
    B, T = ids.shape
    V, E = tok_emb.shape
    L = w_qkv.shape[0]
    nh = 4
    hs = E // nh
    G = 2 if B % 2 == 0 else 1
    bpp = B // G
    M = bpp * T
    bf = jnp.bfloat16

    body = _make_body(L, nh, hs, T, V, E, bpp)

    ids128 = jnp.broadcast_to(ids.reshape(G, M, 1), (G, M, 128))
    qscale = jnp.concatenate(
        [jnp.full((nh * hs, 1), 1.0 / math.sqrt(hs), jnp.float32),
         jnp.ones((2 * nh * hs, 1), jnp.float32)], axis=0)
    wqkv_s = (w_qkv * qscale).astype(bf)

    const2 = lambda c, l: (0, 0)
    perl3 = lambda c, l: (l, 0, 0)

    return pl.pallas_call(
        body,
        out_shape=jax.ShapeDtypeStruct((B, T, V), tok_emb.dtype),
        grid=(G, L),
        in_specs=[
            pl.BlockSpec((None, M, 128), lambda c, l: (c, 0, 0)),
            pl.BlockSpec((V, E), const2),
            pl.BlockSpec((T, E), const2),
            pl.BlockSpec((None, 3 * nh * hs, E), perl3),
            pl.BlockSpec((None, E, nh * hs), perl3),
            pl.BlockSpec((None, 1, E), perl3),
            pl.BlockSpec((None, 1, E), perl3),
            pl.BlockSpec((None, 4 * E, E), perl3),
            pl.BlockSpec((None, 1, 4 * E), perl3),
            pl.BlockSpec((None, E, 4 * E), perl3),
            pl.BlockSpec((None, 1, E), perl3),
            pl.BlockSpec((None, 1, E), perl3),
            pl.BlockSpec((None, 1, E), perl3),
            pl.BlockSpec((V, E), const2),
            pl.BlockSpec((1, V), const2),
        ],
        out_specs=pl.BlockSpec((bpp, T, V), lambda c, l: (c, 0, 0)),
        scratch_shapes=[pltpu.VMEM((M, E), jnp.float32)],
        compiler_params=pltpu.CompilerParams(
            dimension_semantics=("parallel", "arbitrary"),
            vmem_limit_bytes=64 * 1024 * 1024,
        ),
    )(ids128,
      tok_emb.astype(bf), pos_emb,
      wqkv_s, w_proj.astype(bf),
      ln1_g, ln1_b,
      w_ff1.astype(bf), b_ff1, w_ff2.astype(bf), b_ff2,
      ln2_g, ln2_b,
      w_vocab.astype(bf), b_vocab)


# speedup vs baseline: 1.3165x; 1.3165x over previous
import math

import jax
import jax.numpy as jnp
from jax import lax
from jax.experimental import pallas as pl
from jax.experimental.pallas import tpu as pltpu

_C10 = (((1,), (0,)), ((), ()))
_CLAST = (((1,), (1,)), ((), ()))
_CC00 = (((0,), (0,)), ((), ()))
_C01 = (((0,), (1,)), ((), ()))


def _layernorm_f32(h, g, b, eps=1e-5):
    mu = jnp.mean(h, axis=-1, keepdims=True)
    m2 = jnp.mean(h * h, axis=-1, keepdims=True)
    k = lax.rsqrt(m2 - mu * mu + eps)
    kg = k * g
    return h * kg + (b - mu * kg)


def _make_body(n_layers, n_heads, head_size, seq_len, vocab_size, emb_size, bpp):
    L, nh, hs = n_layers, n_heads, head_size
    T, V, E = seq_len, vocab_size, emb_size
    M = bpp * T
    bf = jnp.bfloat16
    f32 = jnp.float32

    def body(ids_ref, te_ref, pe_ref, wqkv_ref, wproj_ref, g1_ref, b1_ref,
             wf1_ref, bf1_ref, wf2_ref, bf2_ref, g2_ref, b2_ref,
             wv_ref, bv_ref, o_ref, x_scr):
        l = pl.program_id(1)

        @pl.when(l == 0)
        def _embed():
            ids_rep = pltpu.repeat(ids_ref[...], V // 128, axis=1)
            col = lax.broadcasted_iota(jnp.int32, (M, V), 1)
            onehot = jnp.where(ids_rep == col, 1.0, 0.0).astype(bf)
            tok = lax.dot_general(onehot, te_ref[...], _C10,
                                  preferred_element_type=f32)
            x_scr[...] = tok + pltpu.repeat(pe_ref[...], bpp, axis=0)

        x32 = x_scr[...]

        ri = lax.broadcasted_iota(jnp.int32, (T, T), 0)
        ci = lax.broadcasted_iota(jnp.int32, (T, T), 1)
        causal = ci <= ri

        xb = x32.astype(bf)
        qkv = lax.dot_general(xb, wqkv_ref[...], _CLAST,
                              preferred_element_type=f32)
        qkvb = qkv.astype(bf)
        onecol = jnp.where(
            lax.broadcasted_iota(jnp.int32, (T, hs), 1) == 0,
            1.0, 0.0).astype(bf)
        rows = []
        for b in range(bpp):
            r0 = b * T
            houts = []
            for h in range(nh):
                q = qkvb[r0:r0 + T, h * hs:(h + 1) * hs]
                k = qkvb[r0:r0 + T, (nh + h) * hs:(nh + h + 1) * hs]
                v = qkvb[r0:r0 + T, (2 * nh + h) * hs:(2 * nh + h + 1) * hs]
                s = lax.dot_general(q, k, _CLAST,
                                    preferred_element_type=f32)
                e = jnp.exp(jnp.where(causal, s, -1e30))
                v_aug = jnp.concatenate([v, onecol], axis=-1)
                pv = lax.dot_general(e.astype(bf), v_aug, _C10,
                                     preferred_element_type=f32)
                den = pv[:, hs:hs + 1]
                houts.append(pv[:, :hs] * pl.reciprocal(den, approx=True))
            rows.append(jnp.concatenate(houts, axis=-1))
        attn = jnp.concatenate(rows, axis=0).astype(bf)
        proj = lax.dot_general(attn, wproj_ref[...], _CLAST,
                               preferred_element_type=f32)
        h1 = _layernorm_f32(x32 + proj, g1_ref[...], b1_ref[...])
        a = lax.dot_general(h1.astype(bf), wf1_ref[...], _CLAST,
                            preferred_element_type=f32)
        ab = jnp.maximum((a + bf1_ref[...]).astype(bf), 0)
        f = lax.dot_general(ab, wf2_ref[...], _CLAST,
                            preferred_element_type=f32)
        f = f + bf2_ref[...]
        x_new = _layernorm_f32(f + x32, g2_ref[...], b2_ref[...])
        x_scr[...] = x_new

        @pl.when(l == L - 1)
        def _logits():
            y = lax.dot_general(x_new.astype(bf), wv_ref[...], _CLAST,
                                preferred_element_type=f32)
            y = (y + bv_ref[...]).astype(o_ref.dtype)
            for b in range(bpp):
                o_ref[b] = y[b * T:(b + 1) * T, :]

    return body


def kernel(ids, tok_emb, pos_emb, w_qkv, w_proj, ln1_g, ln1_b,
           w_ff1, b_ff1, w_ff2, b_ff2, ln2_g, ln2_b, w_vocab, b_vocab):
    B, T = ids.shape
    V, E = tok_emb.shape
    L = w_qkv.shape[0]
    nh = 4
    hs = E // nh
    G = 2 if B % 2 == 0 else 1
    bpp = B // G
    M = bpp * T
    bf = jnp.bfloat16

    body = _make_body(L, nh, hs, T, V, E, bpp)

    ids128 = jnp.broadcast_to(ids.reshape(G, M, 1), (G, M, 128))
    qscale = jnp.concatenate(
        [jnp.full((nh * hs, 1), 1.0 / math.sqrt(hs), jnp.float32),
         jnp.ones((2 * nh * hs, 1), jnp.float32)], axis=0)
    wqkv_s = (w_qkv * qscale).astype(bf)

    const2 = lambda c, l: (0, 0)
    perl3 = lambda c, l: (l, 0, 0)

    return pl.pallas_call(
        body,
        out_shape=jax.ShapeDtypeStruct((B, T, V), tok_emb.dtype),
        grid=(G, L),
        in_specs=[
            pl.BlockSpec((None, M, 128), lambda c, l: (c, 0, 0)),
            pl.BlockSpec((V, E), const2),
            pl.BlockSpec((T, E), const2),
            pl.BlockSpec((None, 3 * nh * hs, E), perl3),
            pl.BlockSpec((None, E, nh * hs), perl3),
            pl.BlockSpec((None, 1, E), perl3),
            pl.BlockSpec((None, 1, E), perl3),
            pl.BlockSpec((None, 4 * E, E), perl3),
            pl.BlockSpec((None, 1, 4 * E), perl3),
            pl.BlockSpec((None, E, 4 * E), perl3),
            pl.BlockSpec((None, 1, E), perl3),
            pl.BlockSpec((None, 1, E), perl3),
            pl.BlockSpec((None, 1, E), perl3),
            pl.BlockSpec((V, E), const2),
            pl.BlockSpec((1, V), const2),
        ],
        out_specs=pl.BlockSpec((bpp, T, V), lambda c, l: (c, 0, 0)),
        scratch_shapes=[pltpu.VMEM((M, E), jnp.float32)],
        compiler_params=pltpu.CompilerParams(
            dimension_semantics=("parallel", "arbitrary"),
            vmem_limit_bytes=64 * 1024 * 1024,
        ),
    )(ids128,
      tok_emb.astype(bf), pos_emb,
      wqkv_s, w_proj.astype(bf),
      ln1_g, ln1_b,
      w_ff1.astype(bf), b_ff1, w_ff2.astype(bf), b_ff2,
      ln2_g, ln2_b,
      w_vocab.astype(bf), b_vocab)
